# baseline (speedup 1.0000x reference)
import jax
import jax.numpy as jnp
from jax.experimental import pallas as pl
from jax.experimental.pallas import tpu as pltpu

_TM = 512
_NB = 4


def _router_kernel(x_hbm, w_ref, probs_hbm, logits_hbm, z_ref,
                   xbuf, pbuf, lbuf, in_sems, pout_sems, lout_sems):
    n = x_hbm.shape[0]
    nchunks = n // _TM

    def in_copy(k, slot):
        return pltpu.make_async_copy(
            x_hbm.at[pl.ds(k * _TM, _TM), :], xbuf.at[slot], in_sems.at[slot])

    def p_copy(k, slot):
        return pltpu.make_async_copy(
            pbuf.at[slot], probs_hbm.at[pl.ds(k * _TM, _TM), :],
            pout_sems.at[slot])

    def l_copy(k, slot):
        return pltpu.make_async_copy(
            lbuf.at[slot], logits_hbm.at[pl.ds(k * _TM, _TM), :],
            lout_sems.at[slot])

    for s in range(_NB):
        in_copy(s, s).start()

    def body(k, z):
        slot = jax.lax.rem(k, _NB)
        in_copy(k, slot).wait()
        logits = jnp.dot(xbuf[slot], w_ref[...],
                         preferred_element_type=jnp.float32)
        m = jnp.max(logits, axis=-1, keepdims=True)
        e = jnp.exp(logits - m)
        ssum = jnp.sum(e, axis=-1, keepdims=True)
        lse = m + jnp.log(ssum)
        z = z + jnp.sum(lse * lse)

        @pl.when(k >= _NB)
        def _():
            p_copy(k - _NB, slot).wait()
            l_copy(k - _NB, slot).wait()

        lbuf[slot] = logits
        pbuf[slot] = e / ssum
        p_copy(k, slot).start()
        l_copy(k, slot).start()

        nk = k + _NB

        @pl.when(nk < nchunks)
        def _():
            in_copy(nk, slot).start()

        return z

    z = jax.lax.fori_loop(0, nchunks, body, 0.0, unroll=False)
    z_ref[...] = jnp.full((1, 1), z, dtype=jnp.float32)

    for s in range(_NB):
        k = nchunks - _NB + s
        p_copy(k, s).wait()
        l_copy(k, s).wait()


def kernel(token_inputs, W, expert_capacity):
    g, t, h = token_inputs.shape
    e = W.shape[1]
    n = g * t
    x = token_inputs.reshape(n, h)
    probs, logits, z = pl.pallas_call(
        _router_kernel,
        in_specs=[
            pl.BlockSpec(memory_space=pl.ANY),
            pl.BlockSpec((h, e), lambda: (0, 0)),
        ],
        out_specs=[
            pl.BlockSpec(memory_space=pl.ANY),
            pl.BlockSpec(memory_space=pl.ANY),
            pl.BlockSpec((1, 1), lambda: (0, 0)),
        ],
        out_shape=[
            jax.ShapeDtypeStruct((n, e), jnp.float32),
            jax.ShapeDtypeStruct((n, e), jnp.float32),
            jax.ShapeDtypeStruct((1, 1), jnp.float32),
        ],
        scratch_shapes=[
            pltpu.VMEM((_NB, _TM, h), jnp.float32),
            pltpu.VMEM((_NB, _TM, e), jnp.float32),
            pltpu.VMEM((_NB, _TM, e), jnp.float32),
            pltpu.SemaphoreType.DMA((_NB,)),
            pltpu.SemaphoreType.DMA((_NB,)),
            pltpu.SemaphoreType.DMA((_NB,)),
        ],
    )(x, W)
    z_loss = z[0, 0] / n
    return probs.reshape(g, t, e), logits.reshape(g, t, e), z_loss
